# baseline (speedup 1.0000x reference)
import functools

import jax
import jax.numpy as jnp
from jax import lax
from jax.experimental import pallas as pl
from jax.experimental.pallas import tpu as pltpu
from jax.experimental.pallas import tpu_sc as plsc

BATCH = 16384
EMB_DIM = 32
LANES = 16
NUM_CORES = 2
NUM_SUBCORES = 16
NUM_WORKERS = NUM_CORES * NUM_SUBCORES
BPW = BATCH // NUM_WORKERS
IDX_CHUNK = 128
NCHUNK = BPW // IDX_CHUNK


def _make_kernel():
    mesh = plsc.VectorSubcoreMesh(core_axis_name="c", subcore_axis_name="s")

    @functools.partial(
        pl.kernel,
        out_type=jax.ShapeDtypeStruct((BATCH,), jnp.float32),
        mesh=mesh,
        compiler_params=pltpu.CompilerParams(
            needs_layout_passes=False, use_tc_tiling_on_sc=False),
        scratch_types=[
            pltpu.VMEM((NCHUNK, IDX_CHUNK), jnp.int32),
            pltpu.VMEM((NCHUNK, IDX_CHUNK), jnp.int32),
            pltpu.VMEM((EMB_DIM, BPW), jnp.float32),
            pltpu.VMEM((EMB_DIM, BPW), jnp.float32),
            pltpu.VMEM((BPW,), jnp.float32),
            pltpu.SemaphoreType.DMA,
            pltpu.SemaphoreType.DMA,
            pltpu.SemaphoreType.DMA,
            pltpu.SemaphoreType.DMA,
        ],
    )
    def cmf_kernel(users_hbm, items_hbm, uembT_hbm, iembT_hbm, out_hbm,
                   uidx_v, iidx_v, urows_v, irows_v, outv,
                   uisem, iisem, usem, isem):
        wid = lax.axis_index("s") * NUM_CORES + lax.axis_index("c")
        base = wid * BPW

        idx_copies = []
        for j in range(NCHUNK):
            idx_copies.append(pltpu.async_copy(
                users_hbm.at[pl.ds(base + j * IDX_CHUNK, IDX_CHUNK)],
                uidx_v.at[j], uisem))
            idx_copies.append(pltpu.async_copy(
                items_hbm.at[pl.ds(base + j * IDX_CHUNK, IDX_CHUNK)],
                iidx_v.at[j], iisem))
        for cp in idx_copies:
            cp.wait()

        gathers = []
        for d in range(EMB_DIM):
            for j in range(NCHUNK):
                gathers.append(pltpu.async_copy(
                    uembT_hbm.at[d].at[uidx_v.at[j]],
                    urows_v.at[d, pl.ds(j * IDX_CHUNK, IDX_CHUNK)], usem))
                gathers.append(pltpu.async_copy(
                    iembT_hbm.at[d].at[iidx_v.at[j]],
                    irows_v.at[d, pl.ds(j * IDX_CHUNK, IDX_CHUNK)], isem))
        for cp in gathers:
            cp.wait()

        def group(g, carry):
            accs = [jnp.zeros((LANES,), jnp.float32) for _ in range(4)]
            for d in range(EMB_DIM):
                u = urows_v[d, pl.ds(g * LANES, LANES)]
                v = irows_v[d, pl.ds(g * LANES, LANES)]
                accs[d % 4] = accs[d % 4] + u * v
            s = (accs[0] + accs[1]) + (accs[2] + accs[3])
            sig = 1.0 / (1.0 + jnp.exp(-s))
            outv[pl.ds(g * LANES, LANES)] = sig
            return carry

        lax.fori_loop(0, BPW // LANES, group, 0)
        pltpu.sync_copy(outv, out_hbm.at[pl.ds(base, BPW)])

    return cmf_kernel


_cmf = _make_kernel()


def kernel(users, items, user_emb, item_emb):
    return _cmf(users, items, user_emb.T, item_emb.T)


# speedup vs baseline: 5.6641x; 5.6641x over previous
import functools

import jax
import jax.numpy as jnp
from jax import lax
from jax.experimental import pallas as pl
from jax.experimental.pallas import tpu as pltpu
from jax.experimental.pallas import tpu_sc as plsc

BATCH = 16384
EMB_DIM = 32
LANES = 16
NUM_CORES = 2
NUM_SUBCORES = 16
NUM_WORKERS = NUM_CORES * NUM_SUBCORES
BPW = BATCH // NUM_WORKERS
IDX_CHUNK = 128
NCHUNK = BPW // IDX_CHUNK


def _make_kernel():
    mesh = plsc.VectorSubcoreMesh(core_axis_name="c", subcore_axis_name="s")

    @functools.partial(
        pl.kernel,
        out_type=jax.ShapeDtypeStruct((BATCH,), jnp.float32),
        mesh=mesh,
        compiler_params=pltpu.CompilerParams(
            needs_layout_passes=False, use_tc_tiling_on_sc=False),
        scratch_types=[
            pltpu.VMEM((NCHUNK, IDX_CHUNK), jnp.int32),
            pltpu.VMEM((NCHUNK, IDX_CHUNK), jnp.int32),
            pltpu.VMEM((BPW, EMB_DIM), jnp.float32),
            pltpu.VMEM((BPW, EMB_DIM), jnp.float32),
            pltpu.VMEM((BPW,), jnp.float32),
            pltpu.SemaphoreType.DMA,
            pltpu.SemaphoreType.DMA,
            [pltpu.SemaphoreType.DMA] * NCHUNK,
            [pltpu.SemaphoreType.DMA] * NCHUNK,
        ],
    )
    def cmf_kernel(users_hbm, items_hbm, uemb_hbm, iemb_hbm, out_hbm,
                   uidx_v, iidx_v, urows_v, irows_v, outv,
                   uisem, iisem, usems, isems):
        wid = lax.axis_index("s") * NUM_CORES + lax.axis_index("c")
        row0 = wid * NCHUNK
        ui_cp = pltpu.async_copy(
            users_hbm.at[pl.ds(row0, NCHUNK)], uidx_v, uisem)
        ii_cp = pltpu.async_copy(
            items_hbm.at[pl.ds(row0, NCHUNK)], iidx_v, iisem)

        ui_cp.wait()
        ucopies = [
            pltpu.async_copy(uemb_hbm.at[uidx_v.at[j]],
                             urows_v.at[pl.ds(j * IDX_CHUNK, IDX_CHUNK)],
                             usems[j])
            for j in range(NCHUNK)
        ]
        ii_cp.wait()
        icopies = [
            pltpu.async_copy(iemb_hbm.at[iidx_v.at[j]],
                             irows_v.at[pl.ds(j * IDX_CHUNK, IDX_CHUNK)],
                             isems[j])
            for j in range(NCHUNK)
        ]

        def group(g, carry):
            rows = g * LANES + lax.iota(jnp.int32, LANES)
            accs = [jnp.zeros((LANES,), jnp.float32) for _ in range(4)]
            for d in range(EMB_DIM):
                cols = jnp.full((LANES,), d, jnp.int32)
                u = plsc.load_gather(urows_v, [rows, cols])
                v = plsc.load_gather(irows_v, [rows, cols])
                accs[d % 4] = accs[d % 4] + u * v
            s = (accs[0] + accs[1]) + (accs[2] + accs[3])
            sig = 1.0 / (1.0 + jnp.exp(-s))
            outv[pl.ds(g * LANES, LANES)] = sig
            return carry

        groups_per_chunk = IDX_CHUNK // LANES
        for j in range(NCHUNK):
            ucopies[j].wait()
            icopies[j].wait()
            lax.fori_loop(j * groups_per_chunk, (j + 1) * groups_per_chunk,
                          group, 0)

        base = wid * BPW
        pltpu.sync_copy(outv, out_hbm.at[pl.ds(base, BPW)])

    return cmf_kernel


_cmf = _make_kernel()


def kernel(users, items, user_emb, item_emb):
    users2 = users.reshape(NUM_WORKERS * NCHUNK, IDX_CHUNK)
    items2 = items.reshape(NUM_WORKERS * NCHUNK, IDX_CHUNK)
    return _cmf(users2, items2, user_emb, item_emb)
